# baseline (speedup 1.0000x reference)
import functools

import jax
import jax.numpy as jnp
from jax import lax
from jax.experimental import pallas as pl
from jax.experimental.pallas import tpu as pltpu
from jax.experimental.pallas import tpu_sc as plsc

NC = 2
NS = 16


def _pre_body(x_ref, wx_ref, ws_ref, b_ref, y_ref, z_ref):
    xv = x_ref[...]
    dn = (((1,), (1,)), ((), ()))
    y_ref[...] = lax.dot_general(xv, wx_ref[...], dn,
                                 preferred_element_type=jnp.float32)
    z_ref[...] = lax.dot_general(xv, ws_ref[...], dn,
                                 preferred_element_type=jnp.float32) + b_ref[...]


def _t4_body(ccle_ref, wc_ref, t_ref):
    dn = (((1,), (1,)), ((), ()))
    t_ref[...] = lax.dot_general(ccle_ref[...], wc_ref[...], dn,
                                 preferred_element_type=jnp.float32)


def _combine_body(acc_ref, z_ref, o_ref):
    av = acc_ref[...]
    o_ref[...] = jnp.maximum(av[0] + av[1] + z_ref[...], 0.0)


def _make_sc_agg(N, OUT, E):
    NW = NC * NS
    EW = E // NW
    CH = 80
    NCH = EW // CH
    G = 80
    NG = N // G
    NGT = (NG + NS - 1) // NS
    assert EW * NW == E and NCH * CH == EW and NG * G == N
    NVR = OUT // 16

    mesh = plsc.VectorSubcoreMesh(core_axis_name="c", subcore_axis_name="s",
                                  num_cores=NC, num_subcores=NS)

    @functools.partial(
        pl.kernel,
        out_type=jax.ShapeDtypeStruct((NC, N, OUT), jnp.float32),
        mesh=mesh,
        scratch_types=[
            pltpu.VMEM_SHARED((N, OUT), jnp.float32),
            pltpu.VMEM((CH,), jnp.int32),
            pltpu.VMEM((CH,), jnp.int32),
            pltpu.VMEM((CH,), jnp.int32),
            pltpu.VMEM((CH,), jnp.float32),
            pltpu.VMEM((CH, OUT), jnp.float32),
            pltpu.VMEM((G, OUT), jnp.float32),
            pltpu.SemaphoreType.DMA,
        ],
    )
    def sc_agg(y_hbm, t4_hbm, ni_hbm, nd_hbm, rel_hbm, w_hbm, acc_out,
               acc_sp, idx_dst, idx_src, rel_v, w_v, rows, zbuf, sem):
        cid = lax.axis_index("c")
        sid = lax.axis_index("s")
        wid = cid * NS + sid

        zero16 = jnp.zeros((16,), jnp.float32)

        def zb_body(r, _):
            for j in range(NVR):
                zbuf[r, pl.ds(j * 16, 16)] = zero16
            return 0
        lax.fori_loop(0, G, zb_body, 0)

        def zero_body(t, _):
            g = sid + t * NS

            @pl.when(g < NG)
            def _():
                pltpu.sync_copy(zbuf, acc_sp.at[pl.ds(g * G, G)])
            return 0
        lax.fori_loop(0, NGT, zero_body, 0)
        plsc.subcore_barrier()

        def chunk_body(c, _):
            off = wid * EW + c * CH
            pltpu.sync_copy(nd_hbm.at[pl.ds(off, CH)], idx_dst)
            pltpu.sync_copy(ni_hbm.at[pl.ds(off, CH)], idx_src)
            pltpu.sync_copy(rel_hbm.at[pl.ds(off, CH)], rel_v)
            pltpu.sync_copy(w_hbm.at[pl.ds(off, CH)], w_v)
            pltpu.async_copy(y_hbm.at[idx_dst], rows, sem).wait()
            pltpu.async_copy(t4_hbm.at[rel_v], rows, sem, add=True).wait()

            def group_body(g, _):
                w16 = w_v[pl.ds(g * 16, 16)]
                for k in range(16):
                    i = g * 16 + k
                    wgt = w16[k]
                    for j in range(NVR):
                        sl = pl.ds(j * 16, 16)
                        rows[i, sl] = rows[i, sl] * wgt
                return 0
            lax.fori_loop(0, CH // 16, group_body, 0)

            pltpu.sync_copy(rows, acc_sp.at[idx_src], add=True)
            return 0
        lax.fori_loop(0, NCH, chunk_body, 0)

        plsc.subcore_barrier()

        def out_body(t, _):
            g = sid + t * NS

            @pl.when(g < NG)
            def _():
                pltpu.sync_copy(acc_sp.at[pl.ds(g * G, G)],
                                acc_out.at[cid, pl.ds(g * G, G)])
            return 0
        lax.fori_loop(0, NGT, out_body, 0)

    return sc_agg


def kernel(x, edge_index, relation, edge_weight, ccle, W_lin, b_lin,
           W_self, b_self):
    N, D = x.shape
    OUT = W_lin.shape[0]
    E = edge_weight.shape[0]
    R = ccle.shape[0]

    W_x = W_lin[:, :D]
    W_c = W_lin[:, D:]
    bias = (b_lin + b_self).reshape(1, OUT)
    node_in = edge_index[0]
    node_out = edge_index[1]

    BP = 1000
    y, z = pl.pallas_call(
        _pre_body,
        grid=(N // BP,),
        in_specs=[
            pl.BlockSpec((BP, D), lambda i: (i, 0)),
            pl.BlockSpec((OUT, D), lambda i: (0, 0)),
            pl.BlockSpec((OUT, D), lambda i: (0, 0)),
            pl.BlockSpec((1, OUT), lambda i: (0, 0)),
        ],
        out_specs=[
            pl.BlockSpec((BP, OUT), lambda i: (i, 0)),
            pl.BlockSpec((BP, OUT), lambda i: (i, 0)),
        ],
        out_shape=[
            jax.ShapeDtypeStruct((N, OUT), jnp.float32),
            jax.ShapeDtypeStruct((N, OUT), jnp.float32),
        ],
    )(x, W_x, W_self, bias)

    t4 = pl.pallas_call(
        _t4_body,
        out_shape=jax.ShapeDtypeStruct((R, OUT), jnp.float32),
    )(ccle, W_c)

    acc = _make_sc_agg(N, OUT, E)(y, t4, node_in, node_out, relation,
                                  edge_weight)

    BC = 1000
    out = pl.pallas_call(
        _combine_body,
        grid=(N // BC,),
        in_specs=[
            pl.BlockSpec((NC, BC, OUT), lambda i: (0, i, 0)),
            pl.BlockSpec((BC, OUT), lambda i: (i, 0)),
        ],
        out_specs=pl.BlockSpec((BC, OUT), lambda i: (i, 0)),
        out_shape=jax.ShapeDtypeStruct((N, OUT), jnp.float32),
    )(acc, z)
    return out


# speedup vs baseline: 14.1762x; 14.1762x over previous
import functools

import jax
import jax.numpy as jnp
from jax import lax
from jax.experimental import pallas as pl
from jax.experimental.pallas import tpu as pltpu
from jax.experimental.pallas import tpu_sc as plsc

NC = 2
NS = 16


def _pre_body(x_ref, wx_ref, ws_ref, b_ref, ccle_ref, wc_ref, yt_ref, z_ref):
    dn = (((1,), (1,)), ((), ()))
    xv = x_ref[...]
    y = lax.dot_general(xv, wx_ref[...], dn,
                        preferred_element_type=jnp.float32)
    t4 = lax.dot_general(ccle_ref[...], wc_ref[...], dn,
                         preferred_element_type=jnp.float32)
    B, OUT = y.shape
    R = t4.shape[0]
    yt = y[:, None, :] + t4[None, :, :]
    yt_ref[...] = yt.reshape(B * R, OUT)
    z_ref[...] = lax.dot_general(xv, ws_ref[...], dn,
                                 preferred_element_type=jnp.float32) + b_ref[...]


def _gidx_body(nd_ref, rel_ref, gi_ref):
    gi_ref[...] = nd_ref[...] * 4 + rel_ref[...]


def _combine_body(acc_ref, z_ref, o_ref):
    av = acc_ref[...]
    o_ref[...] = jnp.maximum(av[0] + av[1] + z_ref[...], 0.0)


def _make_sc_agg(N, OUT, E):
    NW = NC * NS
    EW = E // NW
    CH = 80
    NCH = EW // CH
    G = 40
    NG = N // G
    NGT = (NG + NS - 1) // NS
    assert EW * NW == E and NCH * CH == EW and NG * G == N
    assert NCH % 2 == 1
    NVR = OUT // 16

    mesh = plsc.VectorSubcoreMesh(core_axis_name="c", subcore_axis_name="s",
                                  num_cores=NC, num_subcores=NS)

    @functools.partial(
        pl.kernel,
        out_type=jax.ShapeDtypeStruct((NC, N, OUT), jnp.float32),
        mesh=mesh,
        scratch_types=[
            pltpu.VMEM_SHARED((N, OUT), jnp.float32),
            pltpu.VMEM((EW,), jnp.int32),
            pltpu.VMEM((EW,), jnp.float32),
            pltpu.VMEM((CH,), jnp.int32),
            pltpu.VMEM((CH,), jnp.int32),
            pltpu.VMEM((CH, OUT), jnp.float32),
            pltpu.VMEM((CH, OUT), jnp.float32),
            pltpu.VMEM((G, OUT), jnp.float32),
            pltpu.SemaphoreType.DMA,
            pltpu.SemaphoreType.DMA,
            pltpu.SemaphoreType.DMA,
            pltpu.SemaphoreType.DMA,
            pltpu.SemaphoreType.DMA,
            pltpu.SemaphoreType.DMA,
        ],
    )
    def sc_agg(yt_hbm, gi_hbm, ni_hbm, w_hbm, acc_out,
               acc_sp, gi_v, w_v, ni0, ni1, rows0, rows1, zbuf,
               sem_g0, sem_g1, sem_s0, sem_s1, sem_n0, sem_n1):
        cid = lax.axis_index("c")
        sid = lax.axis_index("s")
        wid = cid * NS + sid
        base = wid * EW
        rows = (rows0, rows1)
        ni = (ni0, ni1)
        sem_g = (sem_g0, sem_g1)
        sem_s = (sem_s0, sem_s1)
        sem_n = (sem_n0, sem_n1)

        zero16 = jnp.zeros((16,), jnp.float32)

        def zb_body(r, _):
            for j in range(NVR):
                zbuf[r, pl.ds(j * 16, 16)] = zero16
            return 0
        lax.fori_loop(0, G, zb_body, 0)

        pltpu.sync_copy(gi_hbm.at[pl.ds(base, EW)], gi_v)
        pltpu.sync_copy(w_hbm.at[pl.ds(base, EW)], w_v)

        def zero_body(t, _):
            g = sid + t * NS

            @pl.when(g < NG)
            def _():
                pltpu.sync_copy(zbuf, acc_sp.at[pl.ds(g * G, G)])
            return 0
        lax.fori_loop(0, NGT, zero_body, 0)
        plsc.subcore_barrier()

        def issue_front(c, b):
            pltpu.async_copy(yt_hbm.at[gi_v.at[pl.ds(c * CH, CH)]],
                             rows[b], sem_g[b])
            pltpu.async_copy(ni_hbm.at[pl.ds(base + c * CH, CH)],
                             ni[b], sem_n[b])

        def wait_front(c, b):
            pltpu.make_async_copy(yt_hbm.at[gi_v.at[pl.ds(c * CH, CH)]],
                                  rows[b], sem_g[b]).wait()
            pltpu.make_async_copy(ni_hbm.at[pl.ds(base + c * CH, CH)],
                                  ni[b], sem_n[b]).wait()

        def issue_scatter(c, b):
            pltpu.async_copy(rows[b], acc_sp.at[ni[b]], sem_s[b], add=True)

        def wait_scatter(c, b):
            pltpu.make_async_copy(rows[b], acc_sp.at[ni[b]], sem_s[b]).wait()

        def compute(c, b):
            rb = rows[b]
            for g in range(CH // 16):
                w16 = w_v[pl.ds(c * CH + g * 16, 16)]
                for k in range(16):
                    i = g * 16 + k
                    wgt = w16[k]
                    for j in range(NVR):
                        sl = pl.ds(j * 16, 16)
                        rb[i, sl] = rb[i, sl] * wgt

        issue_front(0, 0)

        def pair_body(gidx, _):
            for b in range(2):
                c = gidx * 2 + b
                wait_front(c, b)

                @pl.when(c >= 1)
                def _():
                    wait_scatter(c - 1, 1 - b)
                issue_front(c + 1, 1 - b)
                compute(c, b)
                issue_scatter(c, b)
            return 0
        lax.fori_loop(0, (NCH - 1) // 2, pair_body, 0)

        c_last = NCH - 1
        wait_front(c_last, 0)
        wait_scatter(c_last - 1, 1)
        compute(c_last, 0)
        issue_scatter(c_last, 0)
        wait_scatter(c_last, 0)

        plsc.subcore_barrier()

        def out_body(t, _):
            g = sid + t * NS

            @pl.when(g < NG)
            def _():
                pltpu.sync_copy(acc_sp.at[pl.ds(g * G, G)],
                                acc_out.at[cid, pl.ds(g * G, G)])
            return 0
        lax.fori_loop(0, NGT, out_body, 0)

    return sc_agg


def kernel(x, edge_index, relation, edge_weight, ccle, W_lin, b_lin,
           W_self, b_self):
    N, D = x.shape
    OUT = W_lin.shape[0]
    E = edge_weight.shape[0]
    R = ccle.shape[0]

    W_x = W_lin[:, :D]
    W_c = W_lin[:, D:]
    bias = (b_lin + b_self).reshape(1, OUT)
    node_in = edge_index[0]
    node_out = edge_index[1]

    BP = 1000
    yt, z = pl.pallas_call(
        _pre_body,
        grid=(N // BP,),
        in_specs=[
            pl.BlockSpec((BP, D), lambda i: (i, 0)),
            pl.BlockSpec((OUT, D), lambda i: (0, 0)),
            pl.BlockSpec((OUT, D), lambda i: (0, 0)),
            pl.BlockSpec((1, OUT), lambda i: (0, 0)),
            pl.BlockSpec((R, ccle.shape[1]), lambda i: (0, 0)),
            pl.BlockSpec((OUT, ccle.shape[1]), lambda i: (0, 0)),
        ],
        out_specs=[
            pl.BlockSpec((BP * R, OUT), lambda i: (i, 0)),
            pl.BlockSpec((BP, OUT), lambda i: (i, 0)),
        ],
        out_shape=[
            jax.ShapeDtypeStruct((N * R, OUT), jnp.float32),
            jax.ShapeDtypeStruct((N, OUT), jnp.float32),
        ],
    )(x, W_x, W_self, bias, ccle, W_c)

    EB = 2500
    gidx = pl.pallas_call(
        _gidx_body,
        grid=(1,),
        in_specs=[
            pl.BlockSpec((EB, E // EB), lambda i: (0, 0)),
            pl.BlockSpec((EB, E // EB), lambda i: (0, 0)),
        ],
        out_specs=pl.BlockSpec((EB, E // EB), lambda i: (0, 0)),
        out_shape=jax.ShapeDtypeStruct((EB, E // EB), jnp.int32),
    )(node_out.reshape(EB, E // EB), relation.reshape(EB, E // EB))

    acc = _make_sc_agg(N, OUT, E)(yt, gidx.reshape(E), node_in, edge_weight)

    BC = 1000
    out = pl.pallas_call(
        _combine_body,
        grid=(N // BC,),
        in_specs=[
            pl.BlockSpec((NC, BC, OUT), lambda i: (0, i, 0)),
            pl.BlockSpec((BC, OUT), lambda i: (i, 0)),
        ],
        out_specs=pl.BlockSpec((BC, OUT), lambda i: (i, 0)),
        out_shape=jax.ShapeDtypeStruct((N, OUT), jnp.float32),
    )(acc, z)
    return out


# speedup vs baseline: 15.4142x; 1.0873x over previous
import functools

import jax
import jax.numpy as jnp
from jax import lax
from jax.experimental import pallas as pl
from jax.experimental.pallas import tpu as pltpu
from jax.experimental.pallas import tpu_sc as plsc

NC = 2
NS = 16


def _pre_body(x_ref, wx_ref, ws_ref, b_ref, ccle_ref, wc_ref, nd_ref, rel_ref,
              yt_ref, z_ref, gi_ref):
    dn = (((1,), (1,)), ((), ()))
    xv = x_ref[...]
    y = lax.dot_general(xv, wx_ref[...], dn,
                        preferred_element_type=jnp.float32)
    t4 = lax.dot_general(ccle_ref[...], wc_ref[...], dn,
                         preferred_element_type=jnp.float32)
    B, OUT = y.shape
    R = t4.shape[0]
    yt = y[:, None, :] + t4[None, :, :]
    yt_ref[...] = yt.reshape(B * R, OUT)
    z_ref[...] = lax.dot_general(xv, ws_ref[...], dn,
                                 preferred_element_type=jnp.float32) + b_ref[...]
    gi_ref[...] = nd_ref[...] * 4 + rel_ref[...]


def _combine_body(acc_ref, z_ref, o_ref):
    av = acc_ref[...]
    o_ref[...] = jnp.maximum(av[0] + av[1] + z_ref[...], 0.0)


def _make_sc_agg(N, OUT, E):
    NW = NC * NS
    EW = E // NW
    CH = 40
    NCH = EW // CH
    P = 4
    K = 2
    G = 40
    NG = N // G
    NGT = (NG + NS - 1) // NS
    assert EW * NW == E and NCH * CH == EW and NG * G == N and NCH % P == 2
    NVR = OUT // 16

    mesh = plsc.VectorSubcoreMesh(core_axis_name="c", subcore_axis_name="s",
                                  num_cores=NC, num_subcores=NS)

    @functools.partial(
        pl.kernel,
        out_type=jax.ShapeDtypeStruct((NC, N, OUT), jnp.float32),
        mesh=mesh,
        scratch_types=[
            pltpu.VMEM_SHARED((N, OUT), jnp.float32),
            pltpu.VMEM((EW,), jnp.int32),
            pltpu.VMEM((EW,), jnp.float32),
            [pltpu.VMEM((CH,), jnp.int32) for _ in range(P)],
            [pltpu.VMEM((CH, OUT), jnp.float32) for _ in range(P)],
            pltpu.VMEM((G, OUT), jnp.float32),
            [pltpu.SemaphoreType.DMA for _ in range(P)],
            [pltpu.SemaphoreType.DMA for _ in range(P)],
            [pltpu.SemaphoreType.DMA for _ in range(P)],
        ],
    )
    def sc_agg(yt_hbm, gi_hbm, ni_hbm, w_hbm, acc_out,
               acc_sp, gi_v, w_v, ni, rows, zbuf, sem_g, sem_s, sem_n):
        cid = lax.axis_index("c")
        sid = lax.axis_index("s")
        wid = cid * NS + sid
        base = wid * EW

        zero16 = jnp.zeros((16,), jnp.float32)

        def zb_body(r, _):
            for j in range(NVR):
                zbuf[r, pl.ds(j * 16, 16)] = zero16
            return 0
        lax.fori_loop(0, G, zb_body, 0)

        pltpu.sync_copy(gi_hbm.at[pl.ds(base, EW)], gi_v)
        pltpu.sync_copy(w_hbm.at[pl.ds(base, EW)], w_v)

        def zero_body(t, _):
            g = sid + t * NS

            @pl.when(g < NG)
            def _():
                pltpu.sync_copy(zbuf, acc_sp.at[pl.ds(g * G, G)])
            return 0
        lax.fori_loop(0, NGT, zero_body, 0)
        plsc.subcore_barrier()

        def issue_front(c, b):
            pltpu.async_copy(yt_hbm.at[gi_v.at[pl.ds(c * CH, CH)]],
                             rows[b], sem_g[b])
            pltpu.async_copy(ni_hbm.at[pl.ds(base + c * CH, CH)],
                             ni[b], sem_n[b])

        def wait_front(c, b):
            pltpu.make_async_copy(yt_hbm.at[gi_v.at[pl.ds(c * CH, CH)]],
                                  rows[b], sem_g[b]).wait()
            pltpu.make_async_copy(ni_hbm.at[pl.ds(base + c * CH, CH)],
                                  ni[b], sem_n[b]).wait()

        def issue_scatter(c, b):
            pltpu.async_copy(rows[b], acc_sp.at[ni[b]], sem_s[b], add=True)

        def wait_scatter(c, b):
            pltpu.make_async_copy(rows[b], acc_sp.at[ni[b]], sem_s[b]).wait()

        def compute(c, b):
            rb = rows[b]
            for off, k0 in ((0, 0), (16, 0), (24, 8)):
                w16 = w_v[pl.ds(c * CH + off, 16)]
                for k in range(k0, 16):
                    i = off + k
                    wgt = w16[k]
                    for j in range(NVR):
                        sl = pl.ds(j * 16, 16)
                        rb[i, sl] = rb[i, sl] * wgt

        for b in range(K):
            issue_front(b, b)

        def quad_body(q, _):
            for b in range(P):
                c = q * P + b
                wait_front(c, b)

                @pl.when(c >= K)
                def _():
                    wait_scatter(c - K, (b - K) % P)
                issue_front(c + K, (b + K) % P)
                compute(c, b)
                issue_scatter(c, b)
            return 0
        lax.fori_loop(0, (NCH - K) // P, quad_body, 0)

        for t in range(K):
            c = NCH - K + t
            b = c % P
            wait_front(c, b)
            wait_scatter(c - K, (c - K) % P)
            compute(c, b)
            issue_scatter(c, b)
        for t in range(K):
            c = NCH - K + t
            wait_scatter(c, c % P)

        plsc.subcore_barrier()

        def out_body(t, _):
            g = sid + t * NS

            @pl.when(g < NG)
            def _():
                pltpu.sync_copy(acc_sp.at[pl.ds(g * G, G)],
                                acc_out.at[cid, pl.ds(g * G, G)])
            return 0
        lax.fori_loop(0, NGT, out_body, 0)

    return sc_agg


def kernel(x, edge_index, relation, edge_weight, ccle, W_lin, b_lin,
           W_self, b_self):
    N, D = x.shape
    OUT = W_lin.shape[0]
    E = edge_weight.shape[0]
    R = ccle.shape[0]

    W_x = W_lin[:, :D]
    W_c = W_lin[:, D:]
    bias = (b_lin + b_self).reshape(1, OUT)
    node_in = edge_index[0]
    node_out = edge_index[1]

    BP = 1000
    NB = N // BP
    EL = 400
    EB = E // NB // EL
    yt, z, gidx = pl.pallas_call(
        _pre_body,
        grid=(NB,),
        in_specs=[
            pl.BlockSpec((BP, D), lambda i: (i, 0)),
            pl.BlockSpec((OUT, D), lambda i: (0, 0)),
            pl.BlockSpec((OUT, D), lambda i: (0, 0)),
            pl.BlockSpec((1, OUT), lambda i: (0, 0)),
            pl.BlockSpec((R, ccle.shape[1]), lambda i: (0, 0)),
            pl.BlockSpec((OUT, ccle.shape[1]), lambda i: (0, 0)),
            pl.BlockSpec((EB, EL), lambda i: (i, 0)),
            pl.BlockSpec((EB, EL), lambda i: (i, 0)),
        ],
        out_specs=[
            pl.BlockSpec((BP * R, OUT), lambda i: (i, 0)),
            pl.BlockSpec((BP, OUT), lambda i: (i, 0)),
            pl.BlockSpec((EB, EL), lambda i: (i, 0)),
        ],
        out_shape=[
            jax.ShapeDtypeStruct((N * R, OUT), jnp.float32),
            jax.ShapeDtypeStruct((N, OUT), jnp.float32),
            jax.ShapeDtypeStruct((NB * EB, EL), jnp.int32),
        ],
    )(x, W_x, W_self, bias, ccle, W_c,
      node_out.reshape(NB * EB, EL), relation.reshape(NB * EB, EL))

    acc = _make_sc_agg(N, OUT, E)(yt, gidx.reshape(E), node_in, edge_weight)

    BC = 1000
    out = pl.pallas_call(
        _combine_body,
        grid=(N // BC,),
        in_specs=[
            pl.BlockSpec((NC, BC, OUT), lambda i: (0, i, 0)),
            pl.BlockSpec((BC, OUT), lambda i: (i, 0)),
        ],
        out_specs=pl.BlockSpec((BC, OUT), lambda i: (i, 0)),
        out_shape=jax.ShapeDtypeStruct((N, OUT), jnp.float32),
    )(acc, z)
    return out


# speedup vs baseline: 15.4878x; 1.0048x over previous
import functools

import jax
import jax.numpy as jnp
from jax import lax
from jax.experimental import pallas as pl
from jax.experimental.pallas import tpu as pltpu
from jax.experimental.pallas import tpu_sc as plsc

NC = 2
NS = 16


def _pre_body(x_ref, wx_ref, ws_ref, b_ref, ccle_ref, wc_ref, nd_ref, rel_ref,
              yt_ref, z_ref, gi_ref):
    dn = (((1,), (1,)), ((), ()))
    xv = x_ref[...]
    y = lax.dot_general(xv, wx_ref[...], dn,
                        preferred_element_type=jnp.float32)
    t4 = lax.dot_general(ccle_ref[...], wc_ref[...], dn,
                         preferred_element_type=jnp.float32)
    B, OUT = y.shape
    R = t4.shape[0]
    yt = y[:, None, :] + t4[None, :, :]
    yt_ref[...] = yt.reshape(B * R, OUT)
    z_ref[...] = lax.dot_general(xv, ws_ref[...], dn,
                                 preferred_element_type=jnp.float32) + b_ref[...]
    gi_ref[...] = nd_ref[...] * 4 + rel_ref[...]


def _combine_body(acc_ref, z_ref, o_ref):
    av = acc_ref[...]
    o_ref[...] = jnp.maximum(av[0] + av[1] + z_ref[...], 0.0)


def _make_sc_agg(N, OUT, E):
    NW = NC * NS
    EW = E // NW
    CH = 40
    NCH = EW // CH
    P = 4
    K = 3
    G = 16
    NG = N // G
    NGT = (NG + NS - 1) // NS
    assert EW * NW == E and NCH * CH == EW and NG * G == N
    NVR = OUT // 16

    mesh = plsc.VectorSubcoreMesh(core_axis_name="c", subcore_axis_name="s",
                                  num_cores=NC, num_subcores=NS)

    @functools.partial(
        pl.kernel,
        out_type=jax.ShapeDtypeStruct((NC, N, OUT), jnp.float32),
        mesh=mesh,
        scratch_types=[
            pltpu.VMEM_SHARED((N, OUT), jnp.float32),
            pltpu.VMEM((EW,), jnp.int32),
            pltpu.VMEM((EW,), jnp.float32),
            [pltpu.VMEM((CH,), jnp.int32) for _ in range(P)],
            [pltpu.VMEM((CH, OUT), jnp.float32) for _ in range(P)],
            pltpu.VMEM((G, OUT), jnp.float32),
            [pltpu.SemaphoreType.DMA for _ in range(P)],
            [pltpu.SemaphoreType.DMA for _ in range(P)],
            [pltpu.SemaphoreType.DMA for _ in range(P)],
        ],
    )
    def sc_agg(yt_hbm, gi_hbm, ni_hbm, w_hbm, acc_out,
               acc_sp, gi_v, w_v, ni, rows, zbuf, sem_g, sem_s, sem_n):
        cid = lax.axis_index("c")
        sid = lax.axis_index("s")
        wid = cid * NS + sid
        base = wid * EW

        zero16 = jnp.zeros((16,), jnp.float32)

        def zb_body(r, _):
            for j in range(NVR):
                zbuf[r, pl.ds(j * 16, 16)] = zero16
            return 0
        lax.fori_loop(0, G, zb_body, 0)

        pltpu.sync_copy(gi_hbm.at[pl.ds(base, EW)], gi_v)
        pltpu.sync_copy(w_hbm.at[pl.ds(base, EW)], w_v)

        def zero_body(t, _):
            g = sid + t * NS

            @pl.when(g < NG)
            def _():
                pltpu.sync_copy(zbuf, acc_sp.at[pl.ds(g * G, G)])
            return 0
        lax.fori_loop(0, NGT, zero_body, 0)
        plsc.subcore_barrier()

        def issue_front(c, b):
            pltpu.async_copy(yt_hbm.at[gi_v.at[pl.ds(c * CH, CH)]],
                             rows[b], sem_g[b])
            pltpu.async_copy(ni_hbm.at[pl.ds(base + c * CH, CH)],
                             ni[b], sem_n[b])

        def wait_front(c, b):
            pltpu.make_async_copy(yt_hbm.at[gi_v.at[pl.ds(c * CH, CH)]],
                                  rows[b], sem_g[b]).wait()
            pltpu.make_async_copy(ni_hbm.at[pl.ds(base + c * CH, CH)],
                                  ni[b], sem_n[b]).wait()

        def issue_scatter(c, b):
            pltpu.async_copy(rows[b], acc_sp.at[ni[b]], sem_s[b], add=True)

        def wait_scatter(c, b):
            pltpu.make_async_copy(rows[b], acc_sp.at[ni[b]], sem_s[b]).wait()

        def compute(c, b):
            rb = rows[b]
            for off, k0 in ((0, 0), (16, 0), (24, 8)):
                w16 = w_v[pl.ds(c * CH + off, 16)]
                for k in range(k0, 16):
                    i = off + k
                    wgt = w16[k]
                    for j in range(NVR):
                        sl = pl.ds(j * 16, 16)
                        rb[i, sl] = rb[i, sl] * wgt

        for b in range(K):
            issue_front(b, b)

        QUADS = (NCH - K) // P
        C0 = QUADS * P

        def quad_body(q, _):
            for b in range(P):
                c = q * P + b
                wait_front(c, b)

                @pl.when(c >= K)
                def _():
                    wait_scatter(c - K, (b - K) % P)
                issue_front(c + K, (b + K) % P)
                compute(c, b)
                issue_scatter(c, b)
            return 0
        lax.fori_loop(0, QUADS, quad_body, 0)

        for c in range(C0, NCH):
            b = c % P
            wait_front(c, b)
            if c >= K:
                wait_scatter(c - K, (c - K) % P)
            if c + K < NCH:
                issue_front(c + K, (c + K) % P)
            compute(c, b)
            issue_scatter(c, b)
        for c in range(max(NCH - K, 0), NCH):
            wait_scatter(c, c % P)

        plsc.subcore_barrier()

        def out_body(t, _):
            g = sid + t * NS

            @pl.when(g < NG)
            def _():
                pltpu.sync_copy(acc_sp.at[pl.ds(g * G, G)],
                                acc_out.at[cid, pl.ds(g * G, G)])
            return 0
        lax.fori_loop(0, NGT, out_body, 0)

    return sc_agg


def kernel(x, edge_index, relation, edge_weight, ccle, W_lin, b_lin,
           W_self, b_self):
    N, D = x.shape
    OUT = W_lin.shape[0]
    E = edge_weight.shape[0]
    R = ccle.shape[0]

    W_x = W_lin[:, :D]
    W_c = W_lin[:, D:]
    bias = (b_lin + b_self).reshape(1, OUT)
    node_in = edge_index[0]
    node_out = edge_index[1]

    BP = 1000
    NB = N // BP
    EL = 400
    EB = E // NB // EL
    yt, z, gidx = pl.pallas_call(
        _pre_body,
        grid=(NB,),
        in_specs=[
            pl.BlockSpec((BP, D), lambda i: (i, 0)),
            pl.BlockSpec((OUT, D), lambda i: (0, 0)),
            pl.BlockSpec((OUT, D), lambda i: (0, 0)),
            pl.BlockSpec((1, OUT), lambda i: (0, 0)),
            pl.BlockSpec((R, ccle.shape[1]), lambda i: (0, 0)),
            pl.BlockSpec((OUT, ccle.shape[1]), lambda i: (0, 0)),
            pl.BlockSpec((EB, EL), lambda i: (i, 0)),
            pl.BlockSpec((EB, EL), lambda i: (i, 0)),
        ],
        out_specs=[
            pl.BlockSpec((BP * R, OUT), lambda i: (i, 0)),
            pl.BlockSpec((BP, OUT), lambda i: (i, 0)),
            pl.BlockSpec((EB, EL), lambda i: (i, 0)),
        ],
        out_shape=[
            jax.ShapeDtypeStruct((N * R, OUT), jnp.float32),
            jax.ShapeDtypeStruct((N, OUT), jnp.float32),
            jax.ShapeDtypeStruct((NB * EB, EL), jnp.int32),
        ],
    )(x, W_x, W_self, bias, ccle, W_c,
      node_out.reshape(NB * EB, EL), relation.reshape(NB * EB, EL))

    acc = _make_sc_agg(N, OUT, E)(yt, gidx.reshape(E), node_in, edge_weight)

    BC = 1000
    out = pl.pallas_call(
        _combine_body,
        grid=(N // BC,),
        in_specs=[
            pl.BlockSpec((NC, BC, OUT), lambda i: (0, i, 0)),
            pl.BlockSpec((BC, OUT), lambda i: (i, 0)),
        ],
        out_specs=pl.BlockSpec((BC, OUT), lambda i: (i, 0)),
        out_shape=jax.ShapeDtypeStruct((N, OUT), jnp.float32),
    )(acc, z)
    return out
